# Initial kernel scaffold; baseline (speedup 1.0000x reference)
import functools

import jax
import jax.numpy as jnp
from jax import lax
from jax.experimental import pallas as pl
from jax.experimental.pallas import tpu as pltpu
from jax.experimental.pallas import tpu_sc as plsc

B = 1024
SEQ = 77
VOCAB = 49408
DIM = 512
N_CTX1 = 2
N_CTX2 = 4

NUM_CORES = 2
NUM_SUBCORES = 16
NW = NUM_CORES * NUM_SUBCORES
SEQ_PER_W = B // NW
LANES = 16
NCHUNK = DIM // LANES

_mesh = plsc.VectorSubcoreMesh(
    core_axis_name="c", subcore_axis_name="s",
    num_cores=NUM_CORES, num_subcores=NUM_SUBCORES)


@functools.partial(
    pl.kernel,
    out_type=jax.ShapeDtypeStruct((B * SEQ, DIM), jnp.float32),
    mesh=_mesh,
    scratch_types=[
        pltpu.VMEM((SEQ_PER_W, SEQ), jnp.int32),
        pltpu.VMEM((SEQ_PER_W,), jnp.int32),
        pltpu.VMEM((N_CTX1, DIM), jnp.float32),
        pltpu.VMEM((N_CTX2, DIM), jnp.float32),
        pltpu.VMEM((SEQ, DIM), jnp.float32),
        pltpu.SemaphoreType.DMA,
    ],
)
def _sc_prompt_kernel(tokens_hbm, flags_hbm, table_hbm, ctx1_hbm, ctx2_hbm,
                      out_hbm, tok_v, flag_v, ctx1_v, ctx2_v, rows_v, gsem):
    wid = lax.axis_index("s") * NUM_CORES + lax.axis_index("c")
    b0 = wid * SEQ_PER_W

    pltpu.sync_copy(tokens_hbm.at[pl.ds(b0, SEQ_PER_W)], tok_v)
    pltpu.sync_copy(flags_hbm.at[pl.ds(b0, SEQ_PER_W)], flag_v)
    pltpu.sync_copy(ctx1_hbm, ctx1_v)
    pltpu.sync_copy(ctx2_hbm, ctx2_v)

    @pl.loop(0, SEQ_PER_W)
    def _seq(j):
        pltpu.async_copy(table_hbm.at[tok_v.at[j]], rows_v, gsem).wait()

        fvec = plsc.load_gather(flag_v, [jnp.full((LANES,), j, jnp.int32)])
        use1 = fvec == 0
        for c in range(NCHUNK):
            sl = pl.ds(c * LANES, LANES)
            rows_v[1, sl] = jnp.where(use1, ctx1_v[0, sl], ctx2_v[0, sl])
            rows_v[2, sl] = jnp.where(use1, ctx1_v[1, sl], ctx2_v[1, sl])
            rows_v[3, sl] = jnp.where(use1, rows_v[3, sl], ctx2_v[2, sl])
            rows_v[4, sl] = jnp.where(use1, rows_v[4, sl], ctx2_v[3, sl])

        pltpu.sync_copy(rows_v, out_hbm.at[pl.ds((b0 + j) * SEQ, SEQ)])


@jax.jit
def kernel(tokens, cluster_flag, table, ctx1, ctx2):
    out = _sc_prompt_kernel(tokens, cluster_flag, table, ctx1, ctx2)
    return out.reshape(B, SEQ, DIM)


# speedup vs baseline: 1.9926x; 1.9926x over previous
import functools

import jax
import jax.numpy as jnp
from jax import lax
from jax.experimental import pallas as pl
from jax.experimental.pallas import tpu as pltpu
from jax.experimental.pallas import tpu_sc as plsc

B = 1024
SEQ = 77
VOCAB = 49408
DIM = 512
N_CTX1 = 2
N_CTX2 = 4

NUM_CORES = 2
NUM_SUBCORES = 16
NW = NUM_CORES * NUM_SUBCORES
SEQ_PER_W = B // NW
LANES = 16
NCHUNK = DIM // LANES

_mesh = plsc.VectorSubcoreMesh(
    core_axis_name="c", subcore_axis_name="s",
    num_cores=NUM_CORES, num_subcores=NUM_SUBCORES)


@functools.partial(
    pl.kernel,
    out_type=jax.ShapeDtypeStruct((B * SEQ, DIM), jnp.float32),
    mesh=_mesh,
    scratch_types=[
        pltpu.VMEM((SEQ_PER_W, SEQ), jnp.int32),
        pltpu.VMEM((SEQ_PER_W,), jnp.int32),
        pltpu.VMEM((N_CTX1, DIM), jnp.float32),
        pltpu.VMEM((N_CTX2, DIM), jnp.float32),
        pltpu.VMEM((SEQ, DIM), jnp.float32),
        pltpu.SemaphoreType.DMA,
    ],
    compiler_params=pltpu.CompilerParams(
        use_tc_tiling_on_sc=False, needs_layout_passes=False),
)
def _sc_prompt_kernel(tokens_hbm, flags_hbm, table_hbm, ctx1_hbm, ctx2_hbm,
                      out_hbm, tok_v, flag_v, ctx1_v, ctx2_v, rows_v, gsem):
    wid = lax.axis_index("s") * NUM_CORES + lax.axis_index("c")
    b0 = wid * SEQ_PER_W

    pltpu.sync_copy(tokens_hbm.at[pl.ds(b0, SEQ_PER_W)], tok_v)
    pltpu.sync_copy(flags_hbm.at[pl.ds(b0, SEQ_PER_W)], flag_v)
    pltpu.sync_copy(ctx1_hbm, ctx1_v)
    pltpu.sync_copy(ctx2_hbm, ctx2_v)

    @pl.loop(0, SEQ_PER_W)
    def _seq(j):
        pltpu.async_copy(table_hbm.at[tok_v.at[j]], rows_v, gsem).wait()

        fvec = plsc.load_gather(flag_v, [jnp.full((LANES,), j, jnp.int32)])
        use1 = fvec == 0
        for c in range(NCHUNK):
            sl = pl.ds(c * LANES, LANES)
            rows_v[1, sl] = jnp.where(use1, ctx1_v[0, sl], ctx2_v[0, sl])
            rows_v[2, sl] = jnp.where(use1, ctx1_v[1, sl], ctx2_v[1, sl])
            rows_v[3, sl] = jnp.where(use1, rows_v[3, sl], ctx2_v[2, sl])
            rows_v[4, sl] = jnp.where(use1, rows_v[4, sl], ctx2_v[3, sl])

        pltpu.sync_copy(rows_v, out_hbm.at[pl.ds((b0 + j) * SEQ, SEQ)])


@jax.jit
def kernel(tokens, cluster_flag, table, ctx1, ctx2):
    out = _sc_prompt_kernel(tokens, cluster_flag, table, ctx1, ctx2)
    return out.reshape(B, SEQ, DIM)


# speedup vs baseline: 2.0986x; 1.0532x over previous
import functools

import jax
import jax.numpy as jnp
from jax import lax
from jax.experimental import pallas as pl
from jax.experimental.pallas import tpu as pltpu
from jax.experimental.pallas import tpu_sc as plsc

B = 1024
SEQ = 77
VOCAB = 49408
DIM = 512
N_CTX1 = 2
N_CTX2 = 4

NUM_CORES = 2
NUM_SUBCORES = 16
NW = NUM_CORES * NUM_SUBCORES
SEQ_PER_W = B // NW
LANES = 16
NCHUNK = DIM // LANES
NBUF = 2

_mesh = plsc.VectorSubcoreMesh(
    core_axis_name="c", subcore_axis_name="s",
    num_cores=NUM_CORES, num_subcores=NUM_SUBCORES)


@functools.partial(
    pl.kernel,
    out_type=jax.ShapeDtypeStruct((B * SEQ, DIM), jnp.float32),
    mesh=_mesh,
    scratch_types=[
        pltpu.VMEM((SEQ_PER_W, SEQ), jnp.int32),
        pltpu.VMEM((SEQ_PER_W,), jnp.int32),
        pltpu.VMEM((N_CTX1, DIM), jnp.float32),
        pltpu.VMEM((N_CTX2, DIM), jnp.float32),
        pltpu.VMEM((SEQ, DIM), jnp.float32),
        pltpu.VMEM((SEQ, DIM), jnp.float32),
        pltpu.SemaphoreType.DMA,
        pltpu.SemaphoreType.DMA,
        pltpu.SemaphoreType.DMA,
        pltpu.SemaphoreType.DMA,
    ],
    compiler_params=pltpu.CompilerParams(
        use_tc_tiling_on_sc=False, needs_layout_passes=False),
)
def _sc_prompt_kernel(tokens_hbm, flags_hbm, table_hbm, ctx1_hbm, ctx2_hbm,
                      out_hbm, tok_v, flag_v, ctx1_v, ctx2_v,
                      rows0_v, rows1_v, gsem0, gsem1, ssem0, ssem1):
    wid = lax.axis_index("s") * NUM_CORES + lax.axis_index("c")
    b0 = wid * SEQ_PER_W
    rows = (rows0_v, rows1_v)
    gsems = (gsem0, gsem1)
    ssems = (ssem0, ssem1)

    pltpu.sync_copy(tokens_hbm.at[pl.ds(b0, SEQ_PER_W)], tok_v)
    pltpu.sync_copy(flags_hbm.at[pl.ds(b0, SEQ_PER_W)], flag_v)
    pltpu.sync_copy(ctx1_hbm, ctx1_v)
    pltpu.sync_copy(ctx2_hbm, ctx2_v)

    def start_gather(j, slot):
        pltpu.async_copy(table_hbm.at[tok_v.at[j]], rows[slot], gsems[slot])

    def wait_gather(j, slot):
        pltpu.make_async_copy(
            table_hbm.at[tok_v.at[j]], rows[slot], gsems[slot]).wait()

    def start_store(j, slot):
        pltpu.async_copy(
            rows[slot], out_hbm.at[pl.ds((b0 + j) * SEQ, SEQ)], ssems[slot])

    def wait_store(slot):
        pltpu.make_async_copy(
            rows[slot], out_hbm.at[pl.ds(0, SEQ)], ssems[slot]).wait()

    def patch(j, slot):
        buf = rows[slot]
        fvec = plsc.load_gather(flag_v, [jnp.full((LANES,), j, jnp.int32)])
        use1 = fvec == 0
        for c in range(NCHUNK):
            sl = pl.ds(c * LANES, LANES)
            buf[1, sl] = jnp.where(use1, ctx1_v[0, sl], ctx2_v[0, sl])
            buf[2, sl] = jnp.where(use1, ctx1_v[1, sl], ctx2_v[1, sl])
            buf[3, sl] = jnp.where(use1, buf[3, sl], ctx2_v[2, sl])
            buf[4, sl] = jnp.where(use1, buf[4, sl], ctx2_v[3, sl])

    for slot in range(NBUF):
        start_gather(slot, slot)

    @pl.loop(0, SEQ_PER_W, step=NBUF)
    def _block(j0):
        for slot in range(NBUF):
            j = j0 + slot
            wait_gather(j, slot)
            patch(j, slot)
            start_store(j, slot)
        for slot in range(NBUF):
            jn = j0 + NBUF + slot

            @pl.when(jn < SEQ_PER_W)
            def _():
                wait_store(slot)
                start_gather(jn, slot)

    for slot in range(NBUF):
        wait_store(slot)


@jax.jit
def kernel(tokens, cluster_flag, table, ctx1, ctx2):
    out = _sc_prompt_kernel(tokens, cluster_flag, table, ctx1, ctx2)
    return out.reshape(B, SEQ, DIM)


# speedup vs baseline: 2.2099x; 1.0530x over previous
import functools

import jax
import jax.numpy as jnp
from jax import lax
from jax.experimental import pallas as pl
from jax.experimental.pallas import tpu as pltpu
from jax.experimental.pallas import tpu_sc as plsc

B = 1024
SEQ = 77
VOCAB = 49408
DIM = 512
N_CTX1 = 2
N_CTX2 = 4

NUM_CORES = 2
NUM_SUBCORES = 16
NW = NUM_CORES * NUM_SUBCORES
ROWS_PER_W = B * SEQ // NW
CHUNK = 112
NCHUNKS = ROWS_PER_W // CHUNK
SEQ_PER_W = B // NW
NPATCH = SEQ_PER_W * 4
LANES = 16
NBUF = 2

_mesh = plsc.VectorSubcoreMesh(
    core_axis_name="c", subcore_axis_name="s",
    num_cores=NUM_CORES, num_subcores=NUM_SUBCORES)


@functools.partial(
    pl.kernel,
    out_type=jax.ShapeDtypeStruct((B * SEQ, DIM), jnp.float32),
    mesh=_mesh,
    scratch_types=[
        pltpu.VMEM((ROWS_PER_W,), jnp.int32),
        pltpu.VMEM((SEQ_PER_W,), jnp.int32),
        pltpu.VMEM((CHUNK, DIM), jnp.float32),
        pltpu.VMEM((CHUNK, DIM), jnp.float32),
        pltpu.VMEM((CHUNK,), jnp.int32),
        pltpu.VMEM((NPATCH - CHUNK,), jnp.int32),
        pltpu.VMEM((CHUNK,), jnp.int32),
        pltpu.VMEM((NPATCH - CHUNK,), jnp.int32),
        pltpu.SemaphoreType.DMA,
        pltpu.SemaphoreType.DMA,
        pltpu.SemaphoreType.DMA,
        pltpu.SemaphoreType.DMA,
    ],
    compiler_params=pltpu.CompilerParams(needs_layout_passes=False),
)
def _sc_prompt_kernel(tokens_hbm, flags_hbm, table_hbm, ctxcat_hbm,
                      out_hbm, tok_v, flag_v, rows0_v, rows1_v,
                      sidx0_v, sidx1_v, didx0_v, didx1_v,
                      gsem0, gsem1, ssem0, ssem1):
    wid = lax.axis_index("s") * NUM_CORES + lax.axis_index("c")
    r0 = wid * ROWS_PER_W
    b0 = wid * SEQ_PER_W
    rows = (rows0_v, rows1_v)
    gsems = (gsem0, gsem1)
    ssems = (ssem0, ssem1)

    pltpu.sync_copy(tokens_hbm.at[pl.ds(r0, ROWS_PER_W)], tok_v)
    pltpu.sync_copy(flags_hbm.at[pl.ds(b0, SEQ_PER_W)], flag_v)

    def start_gather(k, slot):
        pltpu.async_copy(table_hbm.at[tok_v.at[pl.ds(k * CHUNK, CHUNK)]],
                         rows[slot], gsems[slot])

    def wait_gather(k, slot):
        pltpu.make_async_copy(
            table_hbm.at[tok_v.at[pl.ds(k * CHUNK, CHUNK)]],
            rows[slot], gsems[slot]).wait()

    def start_store(k, slot):
        pltpu.async_copy(
            rows[slot], out_hbm.at[pl.ds(r0 + k * CHUNK, CHUNK)], ssems[slot])

    def wait_store(slot):
        pltpu.make_async_copy(
            rows[slot], out_hbm.at[pl.ds(0, CHUNK)], ssems[slot]).wait()

    for slot in range(NBUF):
        start_gather(slot, slot)

    @pl.loop(0, NCHUNKS, step=NBUF)
    def _block(k0):
        for slot in range(NBUF):
            wait_gather(k0 + slot, slot)
            start_store(k0 + slot, slot)
        for slot in range(NBUF):
            kn = k0 + NBUF + slot

            @pl.when(kn < NCHUNKS)
            def _():
                wait_store(slot)
                start_gather(kn, slot)

    for slot in range(NBUF):
        wait_store(slot)

    lane = jax.lax.iota(jnp.int32, LANES)
    for c in range(NPATCH // LANES):
        p = c * LANES + lane
        jv = p >> 2
        iv = p & 3
        fv = plsc.load_gather(flag_v, [jv])
        use2 = fv != 0
        i_dup = iv & 1
        src = jnp.where(use2, iv + N_CTX1, i_dup)
        dst = (b0 + jv) * SEQ + 1 + jnp.where(use2, iv, i_dup)
        sl = pl.ds((c * LANES) % CHUNK, LANES)
        if c * LANES < CHUNK:
            sidx0_v[sl] = src
            didx0_v[sl] = dst
        else:
            sidx1_v[sl] = src
            didx1_v[sl] = dst

    pltpu.async_copy(ctxcat_hbm.at[sidx0_v], rows0_v, gsem0)
    pltpu.async_copy(ctxcat_hbm.at[sidx1_v],
                     rows1_v.at[pl.ds(0, NPATCH - CHUNK)], gsem1)
    pltpu.make_async_copy(ctxcat_hbm.at[sidx0_v], rows0_v, gsem0).wait()
    pltpu.make_async_copy(ctxcat_hbm.at[sidx1_v],
                          rows1_v.at[pl.ds(0, NPATCH - CHUNK)], gsem1).wait()
    pltpu.async_copy(rows0_v, out_hbm.at[didx0_v], ssem0)
    pltpu.async_copy(rows1_v.at[pl.ds(0, NPATCH - CHUNK)],
                     out_hbm.at[didx1_v], ssem1)
    pltpu.make_async_copy(rows0_v, out_hbm.at[didx0_v], ssem0).wait()
    pltpu.make_async_copy(rows1_v.at[pl.ds(0, NPATCH - CHUNK)],
                          out_hbm.at[didx1_v], ssem1).wait()


@jax.jit
def kernel(tokens, cluster_flag, table, ctx1, ctx2):
    ctxcat = jnp.concatenate([ctx1, ctx2], axis=0)
    out = _sc_prompt_kernel(
        tokens.reshape(-1), cluster_flag, table, ctxcat)
    return out.reshape(B, SEQ, DIM)
